# Initial kernel scaffold; baseline (speedup 1.0000x reference)
import jax
import jax.numpy as jnp
from jax.experimental import pallas as pl


def kernel(adj, attr_vec, gc_W, gc_b, mean_W1, mean_b1, mean_bn_g, mean_bn_b, mean_W2, mean_b2, lv_W1, lv_b1, lv_bn_g, lv_bn_b, lv_W2, lv_b2, dec_W1, dec_b1, dec_bn_g, dec_bn_b, dec_W2, dec_b2, noise):
    raise NotImplementedError("write your pallas kernel here")


# speedup vs baseline: 1.0002x; 1.0002x over previous
import jax
import jax.numpy as jnp
from jax.experimental import pallas as pl
from jax.experimental.pallas import tpu as pltpu

N = 2048
AV = 8
DX = 64
GC = 128
Z = 64
ZOUT = Z + AV
REP = 256

BN_ROWS = 256
NBLK = N // BN_ROWS


def _gcn_kernel(x_ref, gcW0_ref, c0_ref, adj_ref, rinv_ref, gcb_ref,
                mask_ref, x2_ref, support_ref):
    i = pl.program_id(0)

    @pl.when(i == 0)
    def _():
        support_ref[...] = (
            jnp.dot(x_ref[...], gcW0_ref[...], preferred_element_type=jnp.float32)
            + c0_ref[...]
        )

    sup = support_ref[...]
    sup_blk = support_ref[pl.ds(i * BN_ROWS, BN_ROWS), :]
    acc = jnp.dot(adj_ref[...], sup, preferred_element_type=jnp.float32) + sup_blk
    y = acc * rinv_ref[...] + gcb_ref[...]
    y = jnp.maximum(y, 0.0)
    x2_ref[...] = y * mask_ref[...]


def _bn(h, g, b):
    m = jnp.mean(h, axis=0, keepdims=True)
    v = jnp.mean((h - m) * (h - m), axis=0, keepdims=True)
    return (h - m) * jax.lax.rsqrt(v + 1e-5) * g + b


def _head(x2, W1, b1, g, bb, W2, b2):
    h = jnp.dot(x2, W1, preferred_element_type=jnp.float32) + b1
    h = _bn(h, g, bb)
    h = jnp.maximum(h, 0.0)
    return jnp.dot(h, W2, preferred_element_type=jnp.float32) + b2


def _vae_kernel(x2_ref, mW1_ref, mb1_ref, mg_ref, mbb_ref, mW2_ref, mb2_ref,
                lW1_ref, lb1_ref, lg_ref, lbb_ref, lW2_ref, lb2_ref,
                dW1_ref, db1_ref, dg_ref, dbb_ref, dW2_ref, db2_ref,
                noise_ref,
                zmean_ref, zlogvar_ref, h_ref):
    x2 = x2_ref[...]
    z_mean = _head(x2, mW1_ref[...], mb1_ref[...], mg_ref[...], mbb_ref[...],
                   mW2_ref[...], mb2_ref[...])
    z_logvar = _head(x2, lW1_ref[...], lb1_ref[...], lg_ref[...], lbb_ref[...],
                     lW2_ref[...], lb2_ref[...])
    zmean_ref[...] = z_mean
    zlogvar_ref[...] = z_logvar
    z = z_mean + jnp.exp(0.5 * z_logvar) * noise_ref[...]
    hd = jnp.dot(z, dW1_ref[...], preferred_element_type=jnp.float32) + db1_ref[...]
    hd = _bn(hd, dg_ref[...], dbb_ref[...])
    hd = jnp.maximum(hd, 0.0)
    h_ref[...] = jnp.dot(hd, dW2_ref[...], preferred_element_type=jnp.float32) + db2_ref[...]


def _outer_kernel(hblk_ref, hT_ref, out_ref):
    out_ref[...] = jnp.dot(hblk_ref[...], hT_ref[...],
                           preferred_element_type=jnp.float32)


def kernel(adj, attr_vec, gc_W, gc_b, mean_W1, mean_b1, mean_bn_g, mean_bn_b,
           mean_W2, mean_b2, lv_W1, lv_b1, lv_bn_g, lv_bn_b, lv_W2, lv_b2,
           dec_W1, dec_b1, dec_bn_g, dec_bn_b, dec_W2, dec_b2, noise):
    f32 = jnp.float32
    deg = jnp.sum(adj, axis=1)
    L = jnp.diag(deg) - adj
    _, v = jnp.linalg.eigh(L)
    x = v[:, :DX]

    r_inv = (1.0 / (deg + 1.0)).reshape(N, 1)
    drop_mask = jax.random.bernoulli(jax.random.key(42), 0.5, (N, GC))
    mask2 = jnp.where(drop_mask, 2.0, 0.0).astype(f32)

    c0 = (attr_vec[None, :] @ gc_W[DX:, :]).astype(f32)
    db1_eff = (dec_b1 + dec_W1[:, Z:] @ attr_vec)[None, :].astype(f32)

    gcn = pl.pallas_call(
        _gcn_kernel,
        grid=(NBLK,),
        in_specs=[
            pl.BlockSpec((N, DX), lambda i: (0, 0)),
            pl.BlockSpec((DX, GC), lambda i: (0, 0)),
            pl.BlockSpec((1, GC), lambda i: (0, 0)),
            pl.BlockSpec((BN_ROWS, N), lambda i: (i, 0)),
            pl.BlockSpec((BN_ROWS, 1), lambda i: (i, 0)),
            pl.BlockSpec((1, GC), lambda i: (0, 0)),
            pl.BlockSpec((BN_ROWS, GC), lambda i: (i, 0)),
        ],
        out_specs=pl.BlockSpec((BN_ROWS, GC), lambda i: (i, 0)),
        out_shape=jax.ShapeDtypeStruct((N, GC), f32),
        scratch_shapes=[pltpu.VMEM((N, GC), f32)],
    )
    x2 = gcn(x, gc_W[:DX, :], c0, adj, r_inv, gc_b[None, :], mask2)

    full = lambda s: pl.BlockSpec(s, lambda: (0,) * len(s))
    vae = pl.pallas_call(
        _vae_kernel,
        in_specs=[
            full((N, GC)),
            full((GC, GC // 4)), full((1, GC // 4)), full((1, GC // 4)),
            full((1, GC // 4)), full((GC // 4, Z)), full((1, Z)),
            full((GC, GC // 4)), full((1, GC // 4)), full((1, GC // 4)),
            full((1, GC // 4)), full((GC // 4, Z)), full((1, Z)),
            full((Z, REP)), full((1, REP)), full((1, REP)), full((1, REP)),
            full((REP, REP // 4)), full((1, REP // 4)),
            full((N, Z)),
        ],
        out_specs=(full((N, Z)), full((N, Z)), full((N, REP // 4))),
        out_shape=(
            jax.ShapeDtypeStruct((N, Z), f32),
            jax.ShapeDtypeStruct((N, Z), f32),
            jax.ShapeDtypeStruct((N, REP // 4), f32),
        ),
    )
    z_mean, z_logvar, h = vae(
        x2,
        mean_W1.T, mean_b1[None, :], mean_bn_g[None, :], mean_bn_b[None, :],
        mean_W2.T, mean_b2[None, :],
        lv_W1.T, lv_b1[None, :], lv_bn_g[None, :], lv_bn_b[None, :],
        lv_W2.T, lv_b2[None, :],
        dec_W1[:, :Z].T, db1_eff, dec_bn_g[None, :], dec_bn_b[None, :],
        dec_W2.T, dec_b2[None, :],
        noise,
    )

    outer = pl.pallas_call(
        _outer_kernel,
        grid=(NBLK,),
        in_specs=[
            pl.BlockSpec((BN_ROWS, REP // 4), lambda i: (i, 0)),
            pl.BlockSpec((REP // 4, N), lambda i: (0, 0)),
        ],
        out_specs=pl.BlockSpec((BN_ROWS, N), lambda i: (i, 0)),
        out_shape=jax.ShapeDtypeStruct((N, N), f32),
    )
    rec_x = outer(h, h.T)

    return (z_mean, z_logvar, rec_x)
